# baseline (device time: 28609 ns/iter reference)
import jax
import jax.numpy as jnp
from jax import lax
from jax.experimental import pallas as pl
from jax.experimental.pallas import tpu as pltpu

N_DEV = 32
N_STEPS = 5
BLK = 64

BITS_A = (0, 1, 2, 3, 4)
BITS_B = (3, 4, 0, 1, 2)

SQH = 64
C_COLS = SQH * 64
W_COLS = C_COLS + SQH


def kernel(x, Wq, K_ext, V_ext, Wo):
    B, Sq, Dm = x.shape
    _, Skv_loc, Hq, Dh = K_ext.shape
    BH = B * Hq

    def body(x_ref, wq_ref, k_ref, v_ref, wo_ref, out_ref,
             commA, commB, recvA, recvB,
             sendA_sems, recvA_sems, sendB_sems, recvB_sems):
        my = lax.axis_index("i")

        barrier_sem = pltpu.get_barrier_semaphore()
        for k in range(N_STEPS):
            pl.semaphore_signal(
                barrier_sem, inc=1,
                device_id=(my ^ (1 << k),),
                device_id_type=pl.DeviceIdType.MESH,
            )

        xf = x_ref[...].reshape(B * Sq, Dm)
        q = jnp.dot(xf, wq_ref[...], preferred_element_type=jnp.float32)
        q = q.reshape(B, Sq, Hq, Dh)

        qb = lax.broadcasted_iota(jnp.int32, (Sq, Skv_loc), 0) // BLK
        kb = my * (Skv_loc // BLK) + (
            lax.broadcasted_iota(jnp.int32, (Sq, Skv_loc), 1) // BLK
        )
        mask = (qb == kb) | (kb == 0) | ((qb + kb) % 3 == 0)

        k_all = k_ref[...]
        v_all = v_ref[...]
        for b in range(B):
            s = jnp.einsum(
                "ihd,jhd->hij", q[b], k_all[b],
                preferred_element_type=jnp.float32,
            ) * 0.125
            w = jnp.where(mask[None], jnp.exp(s), 0.0)
            l = jnp.sum(w, axis=-1)
            c = jnp.einsum(
                "hij,jhd->hid", w, v_all[b],
                preferred_element_type=jnp.float32,
            )
            r0, r1 = b * Hq, (b + 1) * Hq
            commA[0, r0:r1, :C_COLS] = (
                c[:, :SQH, :].reshape(Hq, C_COLS).astype(jnp.bfloat16)
            )
            commA[0, r0:r1, C_COLS:] = l[:, :SQH].astype(jnp.bfloat16)
            commB[0, r0:r1, :C_COLS] = (
                c[:, SQH:, :].reshape(Hq, C_COLS).astype(jnp.bfloat16)
            )
            commB[0, r0:r1, C_COLS:] = l[:, SQH:].astype(jnp.bfloat16)

        pl.semaphore_wait(barrier_sem, N_STEPS)

        def make(chain_comm, chain_recv, ssems, rsems, bit, step):
            return pltpu.make_async_remote_copy(
                src_ref=chain_comm.at[step],
                dst_ref=chain_recv.at[step],
                send_sem=ssems.at[step],
                recv_sem=rsems.at[step],
                device_id=(my ^ (1 << bit),),
                device_id_type=pl.DeviceIdType.MESH,
            )

        rdmas = []
        rA = make(commA, recvA, sendA_sems, recvA_sems, BITS_A[0], 0)
        rB = make(commB, recvB, sendB_sems, recvB_sems, BITS_B[0], 0)
        rA.start()
        rB.start()
        rdmas += [rA, rB]
        for step in range(N_STEPS):
            rA.wait_recv()
            commA[step + 1] = commA[step] + recvA[step]
            if step + 1 < N_STEPS:
                rA = make(commA, recvA, sendA_sems, recvA_sems,
                          BITS_A[step + 1], step + 1)
                rA.start()
                rdmas.append(rA)
            rB.wait_recv()
            commB[step + 1] = commB[step] + recvB[step]
            if step + 1 < N_STEPS:
                rB = make(commB, recvB, sendB_sems, recvB_sems,
                          BITS_B[step + 1], step + 1)
                rB.start()
                rdmas.append(rB)
        for r in rdmas:
            r.wait_send()

        cA = commA[N_STEPS, :, :C_COLS].reshape(B, Hq, SQH, Dh)
        lA = commA[N_STEPS, :, C_COLS:].reshape(B, Hq, SQH)
        cB = commB[N_STEPS, :, :C_COLS].reshape(B, Hq, SQH, Dh)
        lB = commB[N_STEPS, :, C_COLS:].reshape(B, Hq, SQH)
        for b in range(B):
            top = None
            bot = None
            for h in range(Hq):
                ctx_t = cA[b, h].astype(jnp.float32) * (
                    1.0 / lA[b, h].astype(jnp.float32)[:, None]
                )
                ctx_b = cB[b, h].astype(jnp.float32) * (
                    1.0 / lB[b, h].astype(jnp.float32)[:, None]
                )
                wo_h = wo_ref[h * Dh:(h + 1) * Dh, :]
                pt = jnp.dot(ctx_t, wo_h, preferred_element_type=jnp.float32)
                pb = jnp.dot(ctx_b, wo_h, preferred_element_type=jnp.float32)
                top = pt if top is None else top + pt
                bot = pb if bot is None else bot + pb
            out_ref[b, :SQH, :] = top
            out_ref[b, SQH:, :] = bot

    return pl.pallas_call(
        body,
        out_shape=jax.ShapeDtypeStruct((B, Sq, Dm), jnp.float32),
        in_specs=[pl.BlockSpec(memory_space=pltpu.VMEM)] * 5,
        out_specs=pl.BlockSpec(memory_space=pltpu.VMEM),
        scratch_shapes=[
            pltpu.VMEM((N_STEPS + 1, BH, W_COLS), jnp.bfloat16),
            pltpu.VMEM((N_STEPS + 1, BH, W_COLS), jnp.bfloat16),
            pltpu.VMEM((N_STEPS, BH, W_COLS), jnp.bfloat16),
            pltpu.VMEM((N_STEPS, BH, W_COLS), jnp.bfloat16),
            pltpu.SemaphoreType.DMA((N_STEPS,)),
            pltpu.SemaphoreType.DMA((N_STEPS,)),
            pltpu.SemaphoreType.DMA((N_STEPS,)),
            pltpu.SemaphoreType.DMA((N_STEPS,)),
        ],
        compiler_params=pltpu.CompilerParams(collective_id=0),
    )(x, Wq, K_ext, V_ext, Wo)
